# initial kernel scaffold (unmeasured)
import jax
import jax.numpy as jnp
from jax import lax
from jax.experimental import pallas as pl
from jax.experimental.pallas import tpu as pltpu

N_DEV = 32
M, K, N = 4096, 4096, 8192
KC = K // N_DEV
MT = 512
NB = 1024


def kernel(x, w_mat, scale_x, scale_w):
    sc = jnp.reshape((scale_x * scale_w).astype(jnp.float32), (1, 1))

    def body(x_ref, w_ref, sc_ref, out_ref, xg, wg, stage,
             sx, rx, sw, rw, out_sems):
        me = lax.axis_index("i")
        left = lax.rem(me + N_DEV - 1, N_DEV)
        right = lax.rem(me + 1, N_DEV)

        barrier = pltpu.get_barrier_semaphore()
        pl.semaphore_signal(barrier, inc=1, device_id=(left,),
                            device_id_type=pl.DeviceIdType.MESH)
        pl.semaphore_signal(barrier, inc=1, device_id=(right,),
                            device_id_type=pl.DeviceIdType.MESH)
        pl.semaphore_wait(barrier, 2)

        xg[0, :, :] = x_ref[:, :]
        wg[0, :, :] = w_ref[:, :]

        for h in range(N_DEV - 1):
            rdma_x = pltpu.make_async_remote_copy(
                src_ref=xg.at[h], dst_ref=xg.at[h + 1],
                send_sem=sx.at[h], recv_sem=rx.at[h],
                device_id=(right,), device_id_type=pl.DeviceIdType.MESH)
            rdma_w = pltpu.make_async_remote_copy(
                src_ref=wg.at[h], dst_ref=wg.at[h + 1],
                send_sem=sw.at[h], recv_sem=rw.at[h],
                device_id=(right,), device_id_type=pl.DeviceIdType.MESH)
            rdma_x.start()
            rdma_w.start()
            rdma_x.wait()
            rdma_w.wait()

        s_val = sc_ref[0, 0]
        copies = {}
        bi = 0
        for mi in range(M // MT):
            for ni in range(N // NB):
                m0, n0 = mi * MT, ni * NB

                def step(s, acc, m0=m0, n0=n0):
                    xs = xg[s, m0:m0 + MT, :].astype(jnp.bfloat16)
                    ws = wg[s, :, n0:n0 + NB].astype(jnp.bfloat16)
                    return acc + jnp.dot(
                        xs, ws, preferred_element_type=jnp.float32)

                acc = lax.fori_loop(
                    0, N_DEV, step, jnp.zeros((MT, NB), jnp.float32))

                slot = bi % 2
                if bi >= 2:
                    copies[bi - 2].wait()
                stage[slot, :, :] = acc * s_val
                cp = pltpu.make_async_copy(
                    stage.at[slot],
                    out_ref.at[pl.ds(m0, MT), pl.ds(n0, NB)],
                    out_sems.at[slot])
                cp.start()
                copies[bi] = cp
                bi += 1
        copies[bi - 2].wait()
        copies[bi - 1].wait()

    return pl.pallas_call(
        body,
        out_shape=jax.ShapeDtypeStruct((M, N), jnp.float32),
        in_specs=[
            pl.BlockSpec(memory_space=pltpu.VMEM),
            pl.BlockSpec(memory_space=pltpu.VMEM),
            pl.BlockSpec(memory_space=pltpu.VMEM),
        ],
        out_specs=pl.BlockSpec(memory_space=pltpu.ANY),
        scratch_shapes=[
            pltpu.VMEM((N_DEV, M, KC), jnp.int8),
            pltpu.VMEM((N_DEV, KC, N), jnp.int8),
            pltpu.VMEM((2, MT, NB), jnp.float32),
            pltpu.SemaphoreType.DMA((N_DEV - 1,)),
            pltpu.SemaphoreType.DMA((N_DEV - 1,)),
            pltpu.SemaphoreType.DMA((N_DEV - 1,)),
            pltpu.SemaphoreType.DMA((N_DEV - 1,)),
            pltpu.SemaphoreType.DMA((2,)),
        ],
        compiler_params=pltpu.CompilerParams(collective_id=0),
    )(x, w_mat, sc)


# baseline (device time: 969057 ns/iter reference)
import jax

jax.config.update("jax_compilation_cache_dir", "/tmp/jax_cache")
jax.config.update("jax_persistent_cache_min_compile_time_secs", 0.0)
jax.config.update("jax_persistent_cache_min_entry_size_bytes", 0)

import jax.numpy as jnp
from jax import lax
from jax.experimental import pallas as pl
from jax.experimental.pallas import tpu as pltpu

N_DEV = 32
M, K, N = 4096, 4096, 8192
KC = K // N_DEV
MT = 512
NB = 512
F_HOPS = 16
B_HOPS = 15


def kernel(x, w_mat, scale_x, scale_w):
    sc = jnp.reshape((scale_x * scale_w).astype(jnp.float32), (1, 1))

    def body(x_ref, w_ref, sc_ref, out_ref, xg, wg, stage,
             fsx, frx, fsw, frw, bsx, brx, bsw, brw, out_sems):
        me = lax.axis_index("i")
        left = lax.rem(me + N_DEV - 1, N_DEV)
        right = lax.rem(me + 1, N_DEV)

        barrier = pltpu.get_barrier_semaphore()
        pl.semaphore_signal(barrier, inc=1, device_id=(left,),
                            device_id_type=pl.DeviceIdType.MESH)
        pl.semaphore_signal(barrier, inc=1, device_id=(right,),
                            device_id_type=pl.DeviceIdType.MESH)
        pl.semaphore_wait(barrier, 2)

        xg[0, :, :] = x_ref[:, :]
        wg[0, :, :] = w_ref[:, :]

        sends = []
        for h in range(F_HOPS):
            fx = pltpu.make_async_remote_copy(
                src_ref=xg.at[h], dst_ref=xg.at[h + 1],
                send_sem=fsx.at[h], recv_sem=frx.at[h],
                device_id=(right,), device_id_type=pl.DeviceIdType.MESH)
            fw = pltpu.make_async_remote_copy(
                src_ref=wg.at[h], dst_ref=wg.at[h + 1],
                send_sem=fsw.at[h], recv_sem=frw.at[h],
                device_id=(right,), device_id_type=pl.DeviceIdType.MESH)
            fx.start()
            fw.start()
            sends += [fx, fw]
            if h < B_HOPS:
                s_src = (N_DEV - h) % N_DEV
                bx = pltpu.make_async_remote_copy(
                    src_ref=xg.at[s_src], dst_ref=xg.at[N_DEV - 1 - h],
                    send_sem=bsx.at[h], recv_sem=brx.at[h],
                    device_id=(left,), device_id_type=pl.DeviceIdType.MESH)
                bw = pltpu.make_async_remote_copy(
                    src_ref=wg.at[s_src], dst_ref=wg.at[N_DEV - 1 - h],
                    send_sem=bsw.at[h], recv_sem=brw.at[h],
                    device_id=(left,), device_id_type=pl.DeviceIdType.MESH)
                bx.start()
                bw.start()
                sends += [bx, bw]
                bx.wait_recv()
                bw.wait_recv()
            fx.wait_recv()
            fw.wait_recv()
        for r in sends:
            r.wait_send()

        s_val = sc_ref[0, 0]
        copies = {}
        bi = 0
        for mi in range(M // MT):
            for ni in range(N // NB):
                m0, n0 = mi * MT, ni * NB
                xcat = jnp.concatenate(
                    [xg[s, m0:m0 + MT, :] for s in range(N_DEV)], axis=1
                ).astype(jnp.bfloat16)
                wcat = jnp.concatenate(
                    [wg[s, :, n0:n0 + NB] for s in range(N_DEV)], axis=0
                ).astype(jnp.bfloat16)
                acc = jnp.dot(xcat, wcat, preferred_element_type=jnp.float32)

                slot = bi % 2
                if bi >= 2:
                    copies[bi - 2].wait()
                stage[slot, :, :] = acc * s_val
                cp = pltpu.make_async_copy(
                    stage.at[slot],
                    out_ref.at[pl.ds(m0, MT), pl.ds(n0, NB)],
                    out_sems.at[slot])
                cp.start()
                copies[bi] = cp
                bi += 1
        copies[bi - 2].wait()
        copies[bi - 1].wait()

    return pl.pallas_call(
        body,
        out_shape=jax.ShapeDtypeStruct((M, N), jnp.float32),
        in_specs=[
            pl.BlockSpec(memory_space=pltpu.VMEM),
            pl.BlockSpec(memory_space=pltpu.VMEM),
            pl.BlockSpec(memory_space=pltpu.VMEM),
        ],
        out_specs=pl.BlockSpec(memory_space=pl.ANY),
        scratch_shapes=[
            pltpu.VMEM((N_DEV, M, KC), jnp.int8),
            pltpu.VMEM((N_DEV, KC, N), jnp.int8),
            pltpu.VMEM((2, MT, NB), jnp.float32),
            pltpu.SemaphoreType.DMA((F_HOPS,)),
            pltpu.SemaphoreType.DMA((F_HOPS,)),
            pltpu.SemaphoreType.DMA((F_HOPS,)),
            pltpu.SemaphoreType.DMA((F_HOPS,)),
            pltpu.SemaphoreType.DMA((B_HOPS,)),
            pltpu.SemaphoreType.DMA((B_HOPS,)),
            pltpu.SemaphoreType.DMA((B_HOPS,)),
            pltpu.SemaphoreType.DMA((B_HOPS,)),
            pltpu.SemaphoreType.DMA((2,)),
        ],
        compiler_params=pltpu.CompilerParams(
            collective_id=0, vmem_limit_bytes=64 * 1024 * 1024),
    )(x, w_mat, sc)
